# baseline (speedup 1.0000x reference)
import functools

import jax
import jax.numpy as jnp
from jax import lax
from jax.experimental import pallas as pl
from jax.experimental.pallas import tpu as pltpu
from jax.experimental.pallas import tpu_sc as plsc

EMBED = 128
L = 20
NCLS = 28
NCORES = 2
NSUB = 16
NWORKERS = NCORES * NSUB
GW = 80
WPS = 4
SROWS = GW * WPS
GPS = SROWS // L
LANES = 16


def _sc_gather_pool(table, idx2d, n_rows, batch, woff):
    rows_per_w = n_rows // NWORKERS
    b_per_w = batch // NWORKERS
    nsc = rows_per_w // SROWS
    nwin = rows_per_w // GW
    mesh = plsc.VectorSubcoreMesh(core_axis_name="c", subcore_axis_name="s")

    @functools.partial(
        pl.kernel,
        out_type=jax.ShapeDtypeStruct((batch, EMBED), jnp.float32),
        mesh=mesh,
        scratch_types=[pltpu.VMEM((nwin, GW), jnp.int32)]
        + [pltpu.VMEM((SROWS, EMBED), jnp.float32) for _ in range(2)]
        + [pltpu.VMEM((GPS, EMBED), jnp.float32) for _ in range(2)]
        + [pltpu.SemaphoreType.DMA for _ in range(4)],
    )
    def k(table_hbm, idx_hbm, out_hbm, idx_v, buf0, buf1, ps0, ps1,
          gsem0, gsem1, osem0, osem1):
        bufs, psums = (buf0, buf1), (ps0, ps1)
        gsems, osems = (gsem0, gsem1), (osem0, osem1)
        w = lax.axis_index("s") * NCORES + lax.axis_index("c")
        pltpu.sync_copy(idx_hbm.at[pl.ds(woff + w * nwin, nwin)], idx_v)

        def issue_gathers(sc, p):
            for v in range(WPS):
                pltpu.async_copy(
                    table_hbm.at[idx_v.at[sc * WPS + v]],
                    bufs[p].at[pl.ds(v * GW, GW)], gsems[p])

        def wait_gathers(sc, p):
            for v in range(WPS):
                pltpu.make_async_copy(
                    table_hbm.at[idx_v.at[sc * WPS + v]],
                    bufs[p].at[pl.ds(v * GW, GW)], gsems[p]).wait()

        for p in range(2):
            issue_gathers(p, p)

        @pl.loop(0, nsc // 2)
        def _(jj):
            for p in range(2):
                sc = jj * 2 + p
                wait_gathers(sc, p)

                @pl.when(jj > 0)
                def _():
                    pltpu.make_async_copy(
                        psums[p], out_hbm.at[pl.ds(0, GPS)], osems[p]).wait()

                @pl.loop(0, GPS)
                def _(g):
                    for u in range(EMBED // LANES):
                        sl = pl.ds(u * LANES, LANES)
                        acc = bufs[p][g * L, sl]
                        for l in range(1, L):
                            acc = acc + bufs[p][g * L + l, sl]
                        psums[p][g, sl] = acc

                pltpu.async_copy(
                    psums[p],
                    out_hbm.at[pl.ds(w * b_per_w + sc * GPS, GPS)], osems[p])

                @pl.when(jj < nsc // 2 - 1)
                def _():
                    issue_gathers(sc + 2, p)

        for p in range(2):
            pltpu.make_async_copy(
                psums[p], out_hbm.at[pl.ds(0, GPS)], osems[p]).wait()

    return k(table, idx2d)


def _tc_mlp(pooled_sum, W1s, b1, W2s, b2, out_prev, blk_off, batch_all):
    BB = 512
    part = pooled_sum.shape[0]

    def body(p_ref, w1_ref, b1_ref, w2_ref, b2_ref, prev_ref, o_ref):
        del prev_ref
        pooled = (p_ref[...] * (1.0 / L)).astype(jnp.bfloat16)
        h = jnp.maximum(jnp.dot(pooled, w1_ref[...],
                                preferred_element_type=jnp.float32) + b1_ref[...], 0.0)
        o_ref[...] = jnp.dot(h.astype(jnp.bfloat16), w2_ref[...],
                             preferred_element_type=jnp.float32) + b2_ref[...]

    return pl.pallas_call(
        body,
        grid=(part // BB,),
        in_specs=[
            pl.BlockSpec((BB, EMBED), lambda i: (i, 0)),
            pl.BlockSpec((EMBED, W1s.shape[1]), lambda i: (0, 0)),
            pl.BlockSpec((1, W1s.shape[1]), lambda i: (0, 0)),
            pl.BlockSpec((W1s.shape[1], NCLS), lambda i: (0, 0)),
            pl.BlockSpec((1, NCLS), lambda i: (0, 0)),
            pl.BlockSpec(memory_space=pl.ANY),
        ],
        out_specs=pl.BlockSpec((BB, NCLS), lambda i: (blk_off + i, 0)),
        out_shape=jax.ShapeDtypeStruct((batch_all, NCLS), jnp.float32),
        input_output_aliases={5: 0},
    )(pooled_sum, W1s, b1.reshape(1, -1), W2s, b2.reshape(1, -1), out_prev)


def kernel(x, table, W1, b1, W2, b2):
    batch, seq = x.shape
    splits = (batch * 11 // 16, batch * 5 // 16)
    W1s = W1.astype(jnp.bfloat16)
    W2s = W2.astype(jnp.bfloat16)
    xi = x.astype(jnp.int32)
    out = jnp.zeros((batch, NCLS), jnp.float32)
    base = 0
    for part in splits:
        n_rows = part * seq
        idx2d = xi[base:base + part].reshape(n_rows // GW, GW)
        pooled_sum = _sc_gather_pool(table, idx2d, n_rows, part, 0)
        out = _tc_mlp(pooled_sum, W1s, b1, W2s, b2, out, base // 512, batch)
        base += part
    return out


# speedup vs baseline: 1.0106x; 1.0106x over previous
import functools

import jax
import jax.numpy as jnp
from jax import lax
from jax.experimental import pallas as pl
from jax.experimental.pallas import tpu as pltpu
from jax.experimental.pallas import tpu_sc as plsc

EMBED = 128
L = 20
NCLS = 28
NCORES = 2
NSUB = 16
NWORKERS = NCORES * NSUB
GW = 80
WPS = 4
SROWS = GW * WPS
GPS = SROWS // L
LANES = 16


def _sc_gather_pool(table, idx2d, n_rows, batch, woff):
    rows_per_w = n_rows // NWORKERS
    b_per_w = batch // NWORKERS
    nsc = rows_per_w // SROWS
    nwin = rows_per_w // GW
    mesh = plsc.VectorSubcoreMesh(core_axis_name="c", subcore_axis_name="s")

    @functools.partial(
        pl.kernel,
        out_type=jax.ShapeDtypeStruct((batch, EMBED), jnp.float32),
        mesh=mesh,
        scratch_types=[pltpu.VMEM((nwin, GW), jnp.int32)]
        + [pltpu.VMEM((SROWS, EMBED), jnp.float32) for _ in range(2)]
        + [pltpu.VMEM((GPS, EMBED), jnp.float32) for _ in range(2)]
        + [pltpu.SemaphoreType.DMA for _ in range(4)],
    )
    def k(table_hbm, idx_hbm, out_hbm, idx_v, buf0, buf1, ps0, ps1,
          gsem0, gsem1, osem0, osem1):
        bufs, psums = (buf0, buf1), (ps0, ps1)
        gsems, osems = (gsem0, gsem1), (osem0, osem1)
        w = lax.axis_index("s") * NCORES + lax.axis_index("c")
        pltpu.sync_copy(idx_hbm.at[pl.ds(woff + w * nwin, nwin)], idx_v)

        def issue_gathers(sc, p):
            for v in range(WPS):
                pltpu.async_copy(
                    table_hbm.at[idx_v.at[sc * WPS + v]],
                    bufs[p].at[pl.ds(v * GW, GW)], gsems[p])

        def wait_gathers(sc, p):
            for v in range(WPS):
                pltpu.make_async_copy(
                    table_hbm.at[idx_v.at[sc * WPS + v]],
                    bufs[p].at[pl.ds(v * GW, GW)], gsems[p]).wait()

        for p in range(2):
            issue_gathers(p, p)

        @pl.loop(0, nsc // 2)
        def _(jj):
            for p in range(2):
                sc = jj * 2 + p
                wait_gathers(sc, p)

                @pl.when(jj > 0)
                def _():
                    pltpu.make_async_copy(
                        psums[p], out_hbm.at[pl.ds(0, GPS)], osems[p]).wait()

                @pl.loop(0, GPS)
                def _(g):
                    for u in range(EMBED // LANES):
                        sl = pl.ds(u * LANES, LANES)
                        acc = bufs[p][g * L, sl]
                        for l in range(1, L):
                            acc = acc + bufs[p][g * L + l, sl]
                        psums[p][g, sl] = acc

                pltpu.async_copy(
                    psums[p],
                    out_hbm.at[pl.ds(w * b_per_w + sc * GPS, GPS)], osems[p])

                @pl.when(jj < nsc // 2 - 1)
                def _():
                    issue_gathers(sc + 2, p)

        for p in range(2):
            pltpu.make_async_copy(
                psums[p], out_hbm.at[pl.ds(0, GPS)], osems[p]).wait()

    return k(table, idx2d)


def _tc_mlp(pooled_sum, W1s, b1, W2s, b2, out_prev, blk_off, batch_all):
    BB = 512
    part = pooled_sum.shape[0]

    def body(p_ref, w1_ref, b1_ref, w2_ref, b2_ref, prev_ref, o_ref):
        del prev_ref
        pooled = (p_ref[...] * (1.0 / L)).astype(jnp.bfloat16)
        h = jnp.maximum(jnp.dot(pooled, w1_ref[...],
                                preferred_element_type=jnp.float32) + b1_ref[...], 0.0)
        o_ref[...] = jnp.dot(h.astype(jnp.bfloat16), w2_ref[...],
                             preferred_element_type=jnp.float32) + b2_ref[...]

    return pl.pallas_call(
        body,
        grid=(part // BB,),
        in_specs=[
            pl.BlockSpec((BB, EMBED), lambda i: (i, 0)),
            pl.BlockSpec((EMBED, W1s.shape[1]), lambda i: (0, 0)),
            pl.BlockSpec((1, W1s.shape[1]), lambda i: (0, 0)),
            pl.BlockSpec((W1s.shape[1], NCLS), lambda i: (0, 0)),
            pl.BlockSpec((1, NCLS), lambda i: (0, 0)),
            pl.BlockSpec(memory_space=pl.ANY),
        ],
        out_specs=pl.BlockSpec((BB, NCLS), lambda i: (blk_off + i, 0)),
        out_shape=jax.ShapeDtypeStruct((batch_all, NCLS), jnp.float32),
        input_output_aliases={5: 0},
    )(pooled_sum, W1s, b1.reshape(1, -1), W2s, b2.reshape(1, -1), out_prev)


def kernel(x, table, W1, b1, W2, b2):
    batch, seq = x.shape
    splits = (batch * 3 // 4, batch // 4)
    W1s = W1.astype(jnp.bfloat16)
    W2s = W2.astype(jnp.bfloat16)
    xi = x.astype(jnp.int32)
    out = jnp.zeros((batch, NCLS), jnp.float32)
    base = 0
    for part in splits:
        n_rows = part * seq
        idx2d = xi[base:base + part].reshape(n_rows // GW, GW)
        pooled_sum = _sc_gather_pool(table, idx2d, n_rows, part, 0)
        out = _tc_mlp(pooled_sum, W1s, b1, W2s, b2, out, base // 512, batch)
        base += part
    return out


# speedup vs baseline: 1.0196x; 1.0089x over previous
import functools

import jax
import jax.numpy as jnp
from jax import lax
from jax.experimental import pallas as pl
from jax.experimental.pallas import tpu as pltpu
from jax.experimental.pallas import tpu_sc as plsc

EMBED = 128
L = 20
NCLS = 28
NCORES = 2
NSUB = 16
NWORKERS = NCORES * NSUB
GW = 80
WPS = 4
SROWS = GW * WPS
GPS = SROWS // L
LANES = 16


def _sc_gather_pool(table, idx2d, n_rows, batch, woff):
    rows_per_w = n_rows // NWORKERS
    b_per_w = batch // NWORKERS
    nsc = rows_per_w // SROWS
    nwin = rows_per_w // GW
    mesh = plsc.VectorSubcoreMesh(core_axis_name="c", subcore_axis_name="s")

    @functools.partial(
        pl.kernel,
        out_type=jax.ShapeDtypeStruct((batch, EMBED), jnp.float32),
        mesh=mesh,
        scratch_types=[pltpu.VMEM((nwin, GW), jnp.int32)]
        + [pltpu.VMEM((SROWS, EMBED), jnp.float32) for _ in range(2)]
        + [pltpu.VMEM((GPS, EMBED), jnp.float32) for _ in range(2)]
        + [pltpu.SemaphoreType.DMA for _ in range(4)],
    )
    def k(table_hbm, idx_hbm, out_hbm, idx_v, buf0, buf1, ps0, ps1,
          gsem0, gsem1, osem0, osem1):
        bufs, psums = (buf0, buf1), (ps0, ps1)
        gsems, osems = (gsem0, gsem1), (osem0, osem1)
        w = lax.axis_index("s") * NCORES + lax.axis_index("c")
        pltpu.sync_copy(idx_hbm.at[pl.ds(woff + w * nwin, nwin)], idx_v)

        def issue_gathers(sc, p):
            for v in range(WPS):
                pltpu.async_copy(
                    table_hbm.at[idx_v.at[sc * WPS + v]],
                    bufs[p].at[pl.ds(v * GW, GW)], gsems[p])

        def wait_gathers(sc, p):
            for v in range(WPS):
                pltpu.make_async_copy(
                    table_hbm.at[idx_v.at[sc * WPS + v]],
                    bufs[p].at[pl.ds(v * GW, GW)], gsems[p]).wait()

        for p in range(2):
            issue_gathers(p, p)

        @pl.loop(0, nsc // 2)
        def _(jj):
            for p in range(2):
                sc = jj * 2 + p
                wait_gathers(sc, p)

                @pl.when(jj > 0)
                def _():
                    pltpu.make_async_copy(
                        psums[p], out_hbm.at[pl.ds(0, GPS)], osems[p]).wait()

                @pl.loop(0, GPS)
                def _(g):
                    for u in range(EMBED // LANES):
                        sl = pl.ds(u * LANES, LANES)
                        acc = bufs[p][g * L, sl]
                        for l in range(1, L):
                            acc = acc + bufs[p][g * L + l, sl]
                        psums[p][g, sl] = acc

                pltpu.async_copy(
                    psums[p],
                    out_hbm.at[pl.ds(w * b_per_w + sc * GPS, GPS)], osems[p])

                @pl.when(jj < nsc // 2 - 1)
                def _():
                    issue_gathers(sc + 2, p)

        for p in range(2):
            pltpu.make_async_copy(
                psums[p], out_hbm.at[pl.ds(0, GPS)], osems[p]).wait()

    return k(table, idx2d)


def _tc_mlp(pooled_sum, W1s, b1, W2s, b2, out_prev, blk_off, batch_all):
    BB = 512
    part = pooled_sum.shape[0]

    del out_prev, blk_off, batch_all

    def body(p_ref, w1_ref, b1_ref, w2_ref, b2_ref, o_ref):
        pooled = (p_ref[...] * (1.0 / L)).astype(jnp.bfloat16)
        h = jnp.maximum(jnp.dot(pooled, w1_ref[...],
                                preferred_element_type=jnp.float32) + b1_ref[...], 0.0)
        o_ref[...] = jnp.dot(h.astype(jnp.bfloat16), w2_ref[...],
                             preferred_element_type=jnp.float32) + b2_ref[...]

    return pl.pallas_call(
        body,
        grid=(part // BB,),
        in_specs=[
            pl.BlockSpec((BB, EMBED), lambda i: (i, 0)),
            pl.BlockSpec((EMBED, W1s.shape[1]), lambda i: (0, 0)),
            pl.BlockSpec((1, W1s.shape[1]), lambda i: (0, 0)),
            pl.BlockSpec((W1s.shape[1], NCLS), lambda i: (0, 0)),
            pl.BlockSpec((1, NCLS), lambda i: (0, 0)),
        ],
        out_specs=pl.BlockSpec((BB, NCLS), lambda i: (i, 0)),
        out_shape=jax.ShapeDtypeStruct((part, NCLS), jnp.float32),
    )(pooled_sum, W1s, b1.reshape(1, -1), W2s, b2.reshape(1, -1))


def kernel(x, table, W1, b1, W2, b2):
    batch, seq = x.shape
    splits = (batch * 3 // 4, batch // 4)
    W1s = W1.astype(jnp.bfloat16)
    W2s = W2.astype(jnp.bfloat16)
    xi = x.astype(jnp.int32)
    outs = []
    base = 0
    for part in splits:
        n_rows = part * seq
        idx2d = xi[base:base + part].reshape(n_rows // GW, GW)
        pooled_sum = _sc_gather_pool(table, idx2d, n_rows, part, 0)
        outs.append(_tc_mlp(pooled_sum, W1s, b1, W2s, b2, None, 0, batch))
        base += part
    return jnp.concatenate(outs, axis=0)


# speedup vs baseline: 1.0212x; 1.0016x over previous
import functools

import jax
import jax.numpy as jnp
from jax import lax
from jax.experimental import pallas as pl
from jax.experimental.pallas import tpu as pltpu
from jax.experimental.pallas import tpu_sc as plsc

EMBED = 128
L = 20
NCLS = 28
NCORES = 2
NSUB = 16
NWORKERS = NCORES * NSUB
GW = 80
WPS = 4
SROWS = GW * WPS
GPS = SROWS // L
LANES = 16


def _sc_gather_pool(table, idx2d, n_rows, batch, woff):
    rows_per_w = n_rows // NWORKERS
    b_per_w = batch // NWORKERS
    nsc = rows_per_w // SROWS
    nwin = rows_per_w // GW
    mesh = plsc.VectorSubcoreMesh(core_axis_name="c", subcore_axis_name="s")

    @functools.partial(
        pl.kernel,
        out_type=jax.ShapeDtypeStruct((batch, EMBED), jnp.float32),
        mesh=mesh,
        scratch_types=[pltpu.VMEM((nwin, GW), jnp.int32)]
        + [pltpu.VMEM((SROWS, EMBED), jnp.float32) for _ in range(2)]
        + [pltpu.VMEM((GPS, EMBED), jnp.float32) for _ in range(2)]
        + [pltpu.SemaphoreType.DMA for _ in range(4)],
    )
    def k(table_hbm, idx_hbm, out_hbm, idx_v, buf0, buf1, ps0, ps1,
          gsem0, gsem1, osem0, osem1):
        bufs, psums = (buf0, buf1), (ps0, ps1)
        gsems, osems = (gsem0, gsem1), (osem0, osem1)
        w = lax.axis_index("s") * NCORES + lax.axis_index("c")
        pltpu.sync_copy(idx_hbm.at[pl.ds(woff + w * nwin, nwin)], idx_v)

        def issue_gathers(sc, p):
            for v in range(WPS):
                pltpu.async_copy(
                    table_hbm.at[idx_v.at[sc * WPS + v]],
                    bufs[p].at[pl.ds(v * GW, GW)], gsems[p])

        def wait_gathers(sc, p):
            for v in range(WPS):
                pltpu.make_async_copy(
                    table_hbm.at[idx_v.at[sc * WPS + v]],
                    bufs[p].at[pl.ds(v * GW, GW)], gsems[p]).wait()

        for p in range(2):
            issue_gathers(p, p)

        @pl.loop(0, nsc // 2)
        def _(jj):
            for p in range(2):
                sc = jj * 2 + p
                wait_gathers(sc, p)

                @pl.when(jj > 0)
                def _():
                    pltpu.make_async_copy(
                        psums[p], out_hbm.at[pl.ds(0, GPS)], osems[p]).wait()

                @pl.loop(0, GPS)
                def _(g):
                    for u in range(EMBED // LANES):
                        sl = pl.ds(u * LANES, LANES)
                        acc = bufs[p][g * L, sl]
                        for l in range(1, L):
                            acc = acc + bufs[p][g * L + l, sl]
                        psums[p][g, sl] = acc

                pltpu.async_copy(
                    psums[p],
                    out_hbm.at[pl.ds(w * b_per_w + sc * GPS, GPS)], osems[p])

                @pl.when(jj < nsc // 2 - 1)
                def _():
                    issue_gathers(sc + 2, p)

        for p in range(2):
            pltpu.make_async_copy(
                psums[p], out_hbm.at[pl.ds(0, GPS)], osems[p]).wait()

    return k(table, idx2d)


def _tc_mlp(pooled_sum, W1s, b1, W2s, b2):
    BB = 512
    part = pooled_sum.shape[0]

    def body(p_ref, w1_ref, b1_ref, w2_ref, b2_ref, o_ref):
        pooled = (p_ref[...] * (1.0 / L)).astype(jnp.bfloat16)
        h = jnp.maximum(jnp.dot(pooled, w1_ref[...],
                                preferred_element_type=jnp.float32) + b1_ref[...], 0.0)
        o_ref[...] = jnp.dot(h.astype(jnp.bfloat16), w2_ref[...],
                             preferred_element_type=jnp.float32) + b2_ref[...]

    return pl.pallas_call(
        body,
        grid=(part // BB,),
        in_specs=[
            pl.BlockSpec((BB, EMBED), lambda i: (i, 0)),
            pl.BlockSpec((EMBED, W1s.shape[1]), lambda i: (0, 0)),
            pl.BlockSpec((1, W1s.shape[1]), lambda i: (0, 0)),
            pl.BlockSpec((W1s.shape[1], NCLS), lambda i: (0, 0)),
            pl.BlockSpec((1, NCLS), lambda i: (0, 0)),
        ],
        out_specs=pl.BlockSpec((BB, NCLS), lambda i: (i, 0)),
        out_shape=jax.ShapeDtypeStruct((part, NCLS), jnp.float32),
    )(pooled_sum, W1s, b1.reshape(1, -1), W2s, b2.reshape(1, -1))


def kernel(x, table, W1, b1, W2, b2):
    batch, seq = x.shape
    splits = (batch * 3 // 4, batch // 4)
    W1s = W1.astype(jnp.bfloat16)
    W2s = W2.astype(jnp.bfloat16)
    xi = x.astype(jnp.int32)
    outs = []
    base = 0
    for part in splits:
        n_rows = part * seq
        idx2d = xi[base:base + part].reshape(n_rows // GW, GW)
        pooled_sum = _sc_gather_pool(table, idx2d, n_rows, part, 0)
        outs.append(_tc_mlp(pooled_sum, W1s, b1, W2s, b2))
        base += part
    return jnp.concatenate(outs, axis=0)
